# baseline (speedup 1.0000x reference)
import jax
import jax.numpy as jnp
from jax import lax
from jax.experimental import pallas as pl
from jax.experimental.pallas import tpu as pltpu
from jax.experimental.pallas import tpu_sc as plsc

_N = 10000
_E = 160000
_CHUNK = 128
_B = 104
_NTILES = 16
_NB = 104
_NBP = 97
_NH = 8
_NS = 12
_RPT = 624
_COMB = 104
_TAIL_BASE = _NTILES * _RPT
_TAIL = _N - _TAIL_BASE
_MP_NORM = 0.1

_IV0, _IV1, _PF0, _PF1, _MC0, _MC1 = (i * 16 for i in range(6))


def _process_chunk(chunk, nf_ref, out_ref, acc, rowbuf,
                   srcb, dstb, elb, cfb, pbuf,
                   gsem, ssem, sid, src_hbm, dst_hbm, el_hbm, cf_hbm):
    r0 = 2 * chunk

    def zero_body(t, _):
        z = jnp.zeros((16,), jnp.float32)
        for kk in range(8):
            rowbuf[0, t, pl.ds(kk * 16, 16)] = z
        return 0
    lax.fori_loop(0, _COMB, zero_body, 0)
    zb = rowbuf.at[0].at[pl.ds(0, _COMB)]
    row_base = pl.multiple_of(sid * _RPT, 8)

    def zcopy_body(j, _):
        row0 = pl.multiple_of(row_base + j * _COMB, 8)
        pltpu.sync_copy(zb, acc.at[pl.ds(row0, _COMB)])
        return 0
    lax.fori_loop(0, 6, zcopy_body, 0)

    @pl.when(sid == _NTILES - 1)
    def _():
        pltpu.sync_copy(rowbuf.at[0].at[pl.ds(0, _TAIL)],
                        acc.at[pl.ds(_TAIL_BASE, _TAIL)])

    plsc.subcore_barrier()

    def compute_batch(k, i3):
        def body(e, _):
            elv = jnp.full((16,), elb[k, pl.ds(e, 16)][0], jnp.float32)
            cfv = jnp.full((16,), cfb[k, pl.ds(e, 16)][0], jnp.float32)
            for rr in range(2):
                r = r0 + rr
                w0 = (jnp.exp(elv * pbuf[r, pl.ds(_IV0, 16)])
                      * (cfv * pbuf[r, pl.ds(_PF0, 16)]))
                w1 = (jnp.exp(elv * pbuf[r, pl.ds(_IV1, 16)])
                      * (cfv * pbuf[r, pl.ds(_PF1, 16)]))
                base = rr * 64
                rowbuf[i3, e, pl.ds(base, 16)] = (
                    rowbuf[i3, e, pl.ds(base, 16)] * w0)
                for a in range(1, 4):
                    col = base + a * 16
                    rowbuf[i3, e, pl.ds(col, 16)] = (
                        rowbuf[i3, e, pl.ds(col, 16)] * w1)
            return 0
        lax.fori_loop(0, _B, body, 0)

    def batch_body(k, _):
        i3 = lax.rem(k, 3)
        inx = lax.rem(k + 1, 3)

        @pl.when(k >= 2)
        def _():
            pltpu.make_async_copy(rowbuf.at[0],
                                  acc.at[pl.ds(0, _B)], ssem).wait()

        pltpu.make_async_copy(nf_ref.at[pl.ds(0, _B)],
                              rowbuf.at[i3], gsem).wait()

        @pl.when(k + 1 < _NH)
        def _():
            pltpu.async_copy(nf_ref.at[srcb.at[k + 1]], rowbuf.at[inx], gsem)

        compute_batch(k, i3)
        pltpu.async_copy(rowbuf.at[i3], acc.at[dstb.at[k]], ssem, add=True)
        return 0

    def stage_body(h, _):
        off = pl.multiple_of(h * _NH, 8)
        pltpu.sync_copy(src_hbm.at[pl.ds(off, _NH)], srcb)
        pltpu.sync_copy(dst_hbm.at[pl.ds(off, _NH)], dstb)
        pltpu.sync_copy(el_hbm.at[pl.ds(off, _NH)], elb.at[pl.ds(0, _NH)])
        pltpu.sync_copy(cf_hbm.at[pl.ds(off, _NH)], cfb.at[pl.ds(0, _NH)])
        pltpu.async_copy(nf_ref.at[srcb.at[0]], rowbuf.at[0], gsem)
        lax.fori_loop(0, _NH, batch_body, 0)
        for _ in range(2):
            pltpu.make_async_copy(rowbuf.at[0], acc.at[pl.ds(0, _B)],
                                  ssem).wait()
        return 0

    lax.fori_loop(0, _NS, stage_body, 0)

    pltpu.sync_copy(src_hbm.at[pl.ds(_NS * _NH, _NH)], srcb)
    pltpu.sync_copy(dst_hbm.at[pl.ds(_NS * _NH, _NH)], dstb)
    pltpu.sync_copy(el_hbm.at[pl.ds(_NS * _NH, _NH)], elb.at[pl.ds(0, _NH)])
    pltpu.sync_copy(cf_hbm.at[pl.ds(_NS * _NH, _NH)], cfb.at[pl.ds(0, _NH)])
    pltpu.async_copy(nf_ref.at[srcb.at[0]], rowbuf.at[0], gsem)
    pltpu.make_async_copy(nf_ref.at[pl.ds(0, _B)], rowbuf.at[0], gsem).wait()
    compute_batch(0, 0)
    pltpu.async_copy(rowbuf.at[0], acc.at[dstb.at[0]], ssem, add=True)
    pltpu.make_async_copy(rowbuf.at[0], acc.at[pl.ds(0, _B)], ssem).wait()
    plsc.subcore_barrier()

    def comb_body(t, _):
        for rr in range(2):
            r = r0 + rr
            mc0 = pbuf[r, pl.ds(_MC0, 16)]
            mc1 = pbuf[r, pl.ds(_MC1, 16)]
            for a in range(4):
                col = rr * 64 + a * 16
                nfv = rowbuf[0, t, pl.ds(col, 16)]
                av = rowbuf[1, t, pl.ds(col, 16)]
                mc = mc0 if a == 0 else mc1
                rowbuf[0, t, pl.ds(col, 16)] = nfv * mc + av * _MP_NORM
        return 0

    def combine_block(j, _):
        row0 = pl.multiple_of(row_base + j * _COMB, 8)
        pltpu.sync_copy(nf_ref.at[pl.ds(row0, _COMB)],
                        rowbuf.at[0].at[pl.ds(0, _COMB)])
        pltpu.sync_copy(acc.at[pl.ds(row0, _COMB)],
                        rowbuf.at[1].at[pl.ds(0, _COMB)])
        lax.fori_loop(0, _COMB, comb_body, 0)
        pltpu.sync_copy(rowbuf.at[0].at[pl.ds(0, _COMB)],
                        out_ref.at[pl.ds(row0, _COMB)])
        return 0

    lax.fori_loop(0, 6, combine_block, 0)

    @pl.when(sid == _NTILES - 1)
    def _():
        pltpu.sync_copy(nf_ref.at[pl.ds(_TAIL_BASE, _TAIL)],
                        rowbuf.at[0].at[pl.ds(0, _TAIL)])
        pltpu.sync_copy(acc.at[pl.ds(_TAIL_BASE, _TAIL)],
                        rowbuf.at[1].at[pl.ds(0, _TAIL)])
        lax.fori_loop(0, _TAIL, comb_body, 0)
        pltpu.sync_copy(rowbuf.at[0].at[pl.ds(0, _TAIL)],
                        out_ref.at[pl.ds(_TAIL_BASE, _TAIL)])

    plsc.subcore_barrier()


def _sc_body(src_hbm, dst_hbm, el_hbm, cf_hbm, pp_hbm, nf_hbm,
             out_hbm, acc, rowbuf, srcb, dstb, elb, cfb, pbuf, gsem, ssem):
    cid = lax.axis_index("c")
    sid = lax.axis_index("s")

    pltpu.sync_copy(pp_hbm, pbuf)

    for kc in range(2):
        chunk = cid * 2 + kc
        _process_chunk(chunk, nf_hbm.at[chunk], out_hbm.at[chunk],
                       acc, rowbuf, srcb, dstb, elb, cfb, pbuf, gsem, ssem,
                       sid, src_hbm.at[sid], dst_hbm.at[sid],
                       el_hbm.at[sid], cf_hbm.at[sid])


@jax.jit
def kernel(node_feat, edge_lengths, radial_cutoff_fn, edge_index,
           prefactor, invr0, memory_coef):
    n = node_feat.shape[0]
    nf = jnp.moveaxis(node_feat.reshape(n, 4, _CHUNK), 1, 0)

    ept = _E // _NTILES
    padt = _NB * _B - ept

    def _stage(x):
        return jnp.pad(x.reshape(_NTILES, ept), ((0, 0), (0, padt))).reshape(
            _NTILES, _NB, _B)
    src = _stage(edge_index[0].astype(jnp.int32))
    dst = _stage(edge_index[1].astype(jnp.int32))
    el = _stage(edge_lengths)
    cf = _stage(radial_cutoff_fn)

    pp = jnp.concatenate([-invr0[0], -invr0[1], prefactor[0], prefactor[1],
                          memory_coef[0], memory_coef[1],
                          jnp.zeros((8, 32), jnp.float32)], axis=1)

    mesh = plsc.VectorSubcoreMesh(core_axis_name="c", subcore_axis_name="s")
    run = pl.kernel(
        _sc_body,
        out_type=jax.ShapeDtypeStruct((4, n, _CHUNK), jnp.float32),
        mesh=mesh,
        scratch_types=[
            pltpu.VMEM_SHARED((_N, _CHUNK), jnp.float32),
            pltpu.VMEM((3, _B, _CHUNK), jnp.float32),
            pltpu.VMEM((_NH, _B), jnp.int32),
            pltpu.VMEM((_NH, _B), jnp.int32),
            pltpu.VMEM((_NH + 1, _B), jnp.float32),
            pltpu.VMEM((_NH + 1, _B), jnp.float32),
            pltpu.VMEM((8, 128), jnp.float32),
            pltpu.SemaphoreType.DMA,
            pltpu.SemaphoreType.DMA,
        ],
    )
    out = run(src, dst, el, cf, pp, nf)
    return jnp.transpose(out, (1, 0, 2)).reshape(n, 8, 4, 16)


# speedup vs baseline: 1.3708x; 1.3708x over previous
import jax
import jax.numpy as jnp
from jax import lax
from jax.experimental import pallas as pl
from jax.experimental.pallas import tpu as pltpu
from jax.experimental.pallas import tpu_sc as plsc
import functools

_N = 10000
_E = 160000
_CHUNK = 128
_B = 128
_NTILES = 16
_RPT = 624
_COMB = 104
_TAIL_BASE = _NTILES * _RPT
_TAIL = _N - _TAIL_BASE
_NBATCH = _E // _B
_MP_NORM = 0.1


def _zero_rowbuf(rowbuf):
    def body(t, _):
        z = jnp.zeros((16,), jnp.float32)
        for k in range(8):
            rowbuf[t, pl.ds(k * 16, 16)] = z
        return 0
    lax.fori_loop(0, _B, body, 0)


def _process_chunk(chunk, nf_ref, out_ref, acc, rowbuf, accbuf,
                   srcbuf, dstbuf, elbuf, cfbuf, ivbuf, pfbuf, mcbuf,
                   gsem, src_hbm, dst_hbm, el_hbm, cf_hbm, sid):
    r0 = 2 * chunk

    _zero_rowbuf(rowbuf)
    row_base = sid * _RPT
    for j in range(6):
        pltpu.sync_copy(rowbuf.at[pl.ds(0, _COMB)],
                        acc.at[pl.ds(row_base + j * _COMB, _COMB)])

    @pl.when(sid == _NTILES - 1)
    def _():
        pltpu.sync_copy(rowbuf.at[pl.ds(0, _TAIL)],
                        acc.at[pl.ds(_TAIL_BASE, _TAIL)])

    plsc.subcore_barrier()

    ivs = [[ivbuf[g, r0 + rr] for g in range(2)] for rr in range(2)]
    pfs = [[pfbuf[g, r0 + rr] for g in range(2)] for rr in range(2)]

    def edge_body(e, _):
        elv = jnp.full((16,), elbuf[pl.ds(e, 16)][0], jnp.float32)
        cfv = jnp.full((16,), cfbuf[pl.ds(e, 16)][0], jnp.float32)
        for rr in range(2):
            w0 = jnp.exp(elv * ivs[rr][0]) * (cfv * pfs[rr][0])
            w1 = jnp.exp(elv * ivs[rr][1]) * (cfv * pfs[rr][1])
            base = rr * 64
            rowbuf[e, pl.ds(base, 16)] = rowbuf[e, pl.ds(base, 16)] * w0
            for a in range(1, 4):
                col = base + a * 16
                rowbuf[e, pl.ds(col, 16)] = rowbuf[e, pl.ds(col, 16)] * w1
        return 0

    def batch_body(i, _):
        off = pl.multiple_of((sid + i * _NTILES) * _B, _B)
        pltpu.sync_copy(src_hbm.at[pl.ds(off, _B)], srcbuf)
        pltpu.sync_copy(dst_hbm.at[pl.ds(off, _B)], dstbuf)
        pltpu.sync_copy(el_hbm.at[pl.ds(off, _B)], elbuf.at[pl.ds(0, _B)])
        pltpu.sync_copy(cf_hbm.at[pl.ds(off, _B)], cfbuf.at[pl.ds(0, _B)])
        pltpu.async_copy(nf_ref.at[srcbuf], rowbuf, gsem).wait()
        lax.fori_loop(0, _B, edge_body, 0)
        pltpu.sync_copy(rowbuf, acc.at[dstbuf], add=True)
        return 0

    nb = 78 + jnp.where(sid < 2, 1, 0)
    lax.fori_loop(0, nb, batch_body, 0)
    plsc.subcore_barrier()

    mcvals = []
    for rr in range(2):
        row = []
        for a in range(4):
            g = 0 if a == 0 else 1
            row.append(mcbuf[g, r0 + rr])
        mcvals.append(row)

    def comb_body(t, _):
        for rr in range(2):
            for a in range(4):
                col = rr * 64 + a * 16
                nfv = rowbuf[t, pl.ds(col, 16)]
                av = accbuf[t, pl.ds(col, 16)]
                rowbuf[t, pl.ds(col, 16)] = nfv * mcvals[rr][a] + av * _MP_NORM
        return 0

    def combine(row0, nrows):
        pltpu.sync_copy(nf_ref.at[pl.ds(row0, nrows)], rowbuf.at[pl.ds(0, nrows)])
        pltpu.sync_copy(acc.at[pl.ds(row0, nrows)], accbuf.at[pl.ds(0, nrows)])
        lax.fori_loop(0, nrows, comb_body, 0)
        pltpu.sync_copy(rowbuf.at[pl.ds(0, nrows)], out_ref.at[pl.ds(row0, nrows)])

    for j in range(6):
        combine(row_base + j * _COMB, _COMB)

    @pl.when(sid == _NTILES - 1)
    def _():
        combine(_TAIL_BASE, _TAIL)

    plsc.subcore_barrier()


def _sc_body(src_hbm, dst_hbm, el_hbm, cf_hbm, iv_hbm, pf_hbm, mc_hbm,
             nf0, nf1, nf2, nf3, out_hbm, acc, rowbuf, accbuf,
             srcbuf, dstbuf, elbuf, cfbuf, ivbuf, pfbuf, mcbuf, gsem):
    cid = lax.axis_index("c")
    sid = lax.axis_index("s")

    pltpu.sync_copy(iv_hbm, ivbuf)
    pltpu.sync_copy(pf_hbm, pfbuf)
    pltpu.sync_copy(mc_hbm, mcbuf)

    common = dict(acc=acc, rowbuf=rowbuf, accbuf=accbuf, srcbuf=srcbuf,
                  dstbuf=dstbuf, elbuf=elbuf, cfbuf=cfbuf, ivbuf=ivbuf,
                  pfbuf=pfbuf, mcbuf=mcbuf, gsem=gsem, src_hbm=src_hbm,
                  dst_hbm=dst_hbm, el_hbm=el_hbm, cf_hbm=cf_hbm, sid=sid)

    @pl.when(cid == 0)
    def _():
        _process_chunk(0, nf0, out_hbm.at[0], **common)
        _process_chunk(1, nf1, out_hbm.at[1], **common)

    @pl.when(cid == 1)
    def _():
        _process_chunk(2, nf2, out_hbm.at[2], **common)
        _process_chunk(3, nf3, out_hbm.at[3], **common)


@jax.jit
def kernel(node_feat, edge_lengths, radial_cutoff_fn, edge_index,
           prefactor, invr0, memory_coef):
    n = node_feat.shape[0]
    nfc = node_feat.reshape(n, 4, _CHUNK)
    chunks = [nfc[:, k, :] for k in range(4)]
    src = edge_index[0].astype(jnp.int32)
    dst = edge_index[1].astype(jnp.int32)

    mesh = plsc.VectorSubcoreMesh(core_axis_name="c", subcore_axis_name="s")
    run = pl.kernel(
        _sc_body,
        out_type=jax.ShapeDtypeStruct((4, n, _CHUNK), jnp.float32),
        mesh=mesh,
        scratch_types=[
            pltpu.VMEM_SHARED((_N, _CHUNK), jnp.float32),
            pltpu.VMEM((_B, _CHUNK), jnp.float32),
            pltpu.VMEM((_COMB, _CHUNK), jnp.float32),
            pltpu.VMEM((_B,), jnp.int32),
            pltpu.VMEM((_B,), jnp.int32),
            pltpu.VMEM((_B + 16,), jnp.float32),
            pltpu.VMEM((_B + 16,), jnp.float32),
            pltpu.VMEM((2, 8, 16), jnp.float32),
            pltpu.VMEM((2, 8, 16), jnp.float32),
            pltpu.VMEM((2, 8, 16), jnp.float32),
            pltpu.SemaphoreType.DMA,
        ],
    )
    out = run(src, dst, edge_lengths, radial_cutoff_fn,
              -invr0, prefactor, memory_coef,
              chunks[0], chunks[1], chunks[2], chunks[3])
    return jnp.transpose(out, (1, 0, 2)).reshape(n, 8, 4, 16)
